# baseline (speedup 1.0000x reference)
import functools

import jax
import jax.numpy as jnp
from jax import lax
from jax.experimental import pallas as pl
from jax.experimental.pallas import tpu as pltpu
from jax.experimental.pallas import tpu_sc as plsc

_BINS = 128
_C = 26
_BATCH = 16384
_NC, _NS, _L = 2, 16, 16
_NW = _NC * _NS
_COLS = _BATCH // _NW
_CV = _COLS // _L
_FB = _C * _BINS
_PRIV = 8
_HSTRIDE = _FB + 1
_HWORDS = ((_PRIV * _HSTRIDE + 255) // 256) * 256
_BPS = _FB // _NS
_TROWS = 32
_TABN = _TROWS * _BINS

_MESH = plsc.VectorSubcoreMesh(core_axis_name="c", subcore_axis_name="s")


@functools.partial(
    pl.kernel,
    out_type=jax.ShapeDtypeStruct((_NC * _FB,), jnp.float32),
    mesh=_MESH,
    compiler_params=pltpu.CompilerParams(needs_layout_passes=False),
    scratch_types=[
        pltpu.VMEM((_C, _COLS), jnp.float32),
        pltpu.VMEM((_C, _COLS), jnp.int32),
        pltpu.VMEM((_HWORDS,), jnp.float32),
        pltpu.VMEM((_FB,), jnp.float32),
        pltpu.VMEM_SHARED((_NS * _FB,), jnp.float32),
        pltpu.VMEM((_NS * _BPS,), jnp.float32),
        pltpu.VMEM((_BPS,), jnp.float32),
        pltpu.SemaphoreType.DMA,
    ],
)
def _hist_call(s_hbm, p_hbm, cnt_hbm, s_v, p_v, h_v, r_v, shared, cls_v, o_v, sem):
    cid = lax.axis_index("c")
    sid = lax.axis_index("s")
    wid = cid * _NS + sid
    col0 = wid * _COLS
    h_s = pltpu.async_copy(s_hbm.at[:, pl.ds(col0, _COLS)], s_v, sem)
    h_p = pltpu.async_copy(p_hbm.at[:, pl.ds(col0, _COLS)], p_v, sem)

    zero = jnp.zeros((_L,), jnp.float32)

    @plsc.parallel_loop(0, _HWORDS // 256, 1, unroll=2)
    def zbody(i):
        b = i * 256
        for k in range(16):
            h_v[pl.ds(b + k * _L, _L)] = zero

    h_s.wait()
    h_p.wait()

    lane = lax.broadcasted_iota(jnp.int32, (_L,), 0)
    lane_off = (lane % _PRIV) * _HSTRIDE
    mlow = lane < _PRIV
    mhigh = jnp.logical_not(mlow)

    @plsc.parallel_loop(0, _CV, 1, unroll=2)
    def mbody(j):
        b = j * _L
        for c0 in range(0, _C, 8):
            cg = range(c0, min(c0 + 8, _C))
            ss = [s_v[c, pl.ds(b, _L)] for c in cg]
            pp = [p_v[c, pl.ds(b, _L)] for c in cg]
            idxs = [lane_off +
                    (jnp.minimum((s * 128.0).astype(jnp.int32), _BINS - 1)
                     + c * _BINS)
                    for c, s in zip(cg, ss)]
            vals = [p.astype(jnp.float32) for p in pp]
            for idx, val in zip(idxs, vals):
                plsc.addupdate_scatter(h_v, [idx], val, mask=mlow)
                plsc.addupdate_scatter(h_v, [idx], val, mask=mhigh)

    @plsc.parallel_loop(0, _FB // _L, 1, unroll=4)
    def rbody(j):
        b = j * _L
        acc = h_v[pl.ds(b, _L)]
        for l in range(1, _PRIV):
            acc = acc + h_v[pl.ds(l * _HSTRIDE + b, _L)]
        r_v[pl.ds(b, _L)] = acc

    pltpu.sync_copy(r_v, shared.at[pl.ds(sid * _FB, _FB)])
    plsc.subcore_barrier()
    handles = [
        pltpu.async_copy(shared.at[pl.ds(l * _FB + sid * _BPS, _BPS)],
                         cls_v.at[pl.ds(l * _BPS, _BPS)], sem)
        for l in range(_NS)
    ]
    for h in handles:
        h.wait()

    @plsc.parallel_loop(0, _BPS // _L, 1, unroll=2)
    def cbody(k):
        b = k * _L
        acc = cls_v[pl.ds(b, _L)]
        for l in range(1, _NS):
            acc = acc + cls_v[pl.ds(l * _BPS + b, _L)]
        o_v[pl.ds(b, _L)] = acc
    pltpu.sync_copy(o_v, cnt_hbm.at[pl.ds(cid * _FB + sid * _BPS, _BPS)])


def _fit_kernel(cnt_ref, w1_ref, b1_ref, w2_ref, b2_ref, ta_ref, tb_ref):
    cnt2 = jnp.reshape(cnt_ref[...], (2 * _C, _BINS))
    cnt = cnt2[0:_C] + cnt2[_C:2 * _C]
    total = jnp.sum(cnt, axis=1, keepdims=True)
    hist = cnt / total
    h = jnp.clip(hist, 1e-6, 1.0 - 1e-6)
    h = jnp.log(h / (1.0 - h))
    h = lax.dot_general(h, w1_ref[...], (((1,), (1,)), ((), ())),
                        precision=lax.Precision.HIGHEST,
                        preferred_element_type=jnp.float32) \
        + jnp.reshape(b1_ref[...], (1, _BINS))
    h = jnp.where(h >= 0.0, h, 0.01 * h)
    d = lax.dot_general(h, w2_ref[...], (((1,), (1,)), ((), ())),
                        precision=lax.Precision.HIGHEST,
                        preferred_element_type=jnp.float32) \
        + jnp.reshape(b2_ref[...], (1, _BINS))
    mx = jnp.max(d, axis=1, keepdims=True)
    e = jnp.exp(d - mx)
    p = e / jnp.sum(e, axis=1, keepdims=True)
    rr = lax.broadcasted_iota(jnp.int32, (_BINS, _BINS), 0)
    cc = lax.broadcasted_iota(jnp.int32, (_BINS, _BINS), 1)
    tri = (rr <= cc).astype(jnp.float32)
    y = lax.dot_general(p, tri, (((1,), (0,)), ((), ())),
                        precision=lax.Precision.HIGHEST,
                        preferred_element_type=jnp.float32)
    e0 = y - p
    ji = lax.broadcasted_iota(jnp.int32, (1, _BINS), 1)
    j = ji.astype(jnp.float32)
    dxinv = jnp.where(ji == 0, 256.0, 128.0)
    x0 = jnp.where(ji == 0, 0.0, (2.0 * j - 1.0) / 256.0)
    bt = p * dxinv
    at = e0 - bt * x0
    zrows = jnp.zeros((_TROWS - _C, _BINS), jnp.float32)
    ta_ref[0:_C, :] = at
    ta_ref[_C:_TROWS, :] = zrows
    tb_ref[0:_C, :] = bt
    tb_ref[_C:_TROWS, :] = zrows


_fit_call = pl.pallas_call(
    _fit_kernel,
    out_shape=(
        jax.ShapeDtypeStruct((_TROWS, _BINS), jnp.float32),
        jax.ShapeDtypeStruct((_TROWS, _BINS), jnp.float32),
    ),
)


@functools.partial(
    pl.kernel,
    out_type=jax.ShapeDtypeStruct((_C, _BATCH), jnp.float32),
    mesh=_MESH,
    compiler_params=pltpu.CompilerParams(needs_layout_passes=False),
    scratch_types=[
        pltpu.VMEM((_C, _COLS), jnp.float32),
        pltpu.VMEM((_C, _COLS), jnp.int32),
        pltpu.VMEM((_TABN,), jnp.float32),
        pltpu.VMEM((_TABN,), jnp.float32),
        pltpu.VMEM((_C, _COLS), jnp.float32),
        pltpu.SemaphoreType.DMA,
    ],
)
def _interp_call(s_hbm, p_hbm, ta_hbm, tb_hbm, out_hbm,
                 s_v, p_v, ta_v, tb_v, o_v, sem):
    cid = lax.axis_index("c")
    sid = lax.axis_index("s")
    wid = cid * _NS + sid
    col0 = wid * _COLS

    handles = [
        pltpu.async_copy(ta_hbm, ta_v, sem),
        pltpu.async_copy(tb_hbm, tb_v, sem),
        pltpu.async_copy(s_hbm.at[:, pl.ds(col0, _COLS)], s_v, sem),
        pltpu.async_copy(p_hbm.at[:, pl.ds(col0, _COLS)], p_v, sem),
    ]
    for h in handles:
        h.wait()

    ones = jnp.ones((_L,), jnp.float32)

    @plsc.parallel_loop(0, _CV, 1, unroll=2)
    def mbody(j):
        b = j * _L
        for c0 in range(0, _C, 8):
            cg = range(c0, min(c0 + 8, _C))
            ss = [s_v[c, pl.ds(b, _L)] for c in cg]
            pp = [p_v[c, pl.ds(b, _L)] for c in cg]
            iraws = [(s * 128.0 + 0.5).astype(jnp.int32) for s in ss]
            idxs = [jnp.minimum(ir, _BINS - 1) + c * _BINS
                    for c, ir in zip(cg, iraws)]
            aa = [plsc.load_gather(ta_v, [idx]) for idx in idxs]
            bbs = [plsc.load_gather(tb_v, [idx]) for idx in idxs]
            for c, s16, p16, iraw, a, bb in zip(cg, ss, pp, iraws, aa, bbs):
                w = a + bb * s16
                y127 = a + bb * (255.0 / 256.0)
                wedge = y127 + (1.0 - y127) * (256.0 * s16 - 255.0)
                w = jnp.where(iraw >= _BINS, wedge, w)
                o_v[c, pl.ds(b, _L)] = jnp.where(p16 == 1, w, ones)
    pltpu.sync_copy(o_v, out_hbm.at[:, pl.ds(col0, _COLS)])


def kernel(y_score, y_partial, W1, b1, W2, b2):
    s_t = y_score.T
    p_t = y_partial.astype(jnp.int32).T
    cnt = _hist_call(s_t, p_t)
    ta, tb = _fit_call(cnt, W1, b1, W2, b2)
    out_t = _interp_call(s_t, p_t, ta.reshape(_TABN), tb.reshape(_TABN))
    return out_t.T


# speedup vs baseline: 1.0051x; 1.0051x over previous
import functools

import jax
import jax.numpy as jnp
from jax import lax
from jax.experimental import pallas as pl
from jax.experimental.pallas import tpu as pltpu
from jax.experimental.pallas import tpu_sc as plsc

_BINS = 128
_C = 26
_BATCH = 16384
_NC, _NS, _L = 2, 16, 16
_NW = _NC * _NS
_COLS = _BATCH // _NW
_CV = _COLS // _L
_FB = _C * _BINS
_PRIV = 8
_HSTRIDE = _FB + 1
_HWORDS = ((_PRIV * _HSTRIDE + 255) // 256) * 256
_BPS = _FB // _NS
_TROWS = 32
_TABN = _TROWS * _BINS

_MESH = plsc.VectorSubcoreMesh(core_axis_name="c", subcore_axis_name="s")


@functools.partial(
    pl.kernel,
    out_type=jax.ShapeDtypeStruct((_NC * _FB,), jnp.float32),
    mesh=_MESH,
    compiler_params=pltpu.CompilerParams(needs_layout_passes=False),
    scratch_types=[
        pltpu.VMEM((_C, _COLS), jnp.float32),
        pltpu.VMEM((_C, _COLS), jnp.int32),
        pltpu.VMEM((_HWORDS,), jnp.float32),
        pltpu.VMEM((_FB,), jnp.float32),
        pltpu.VMEM_SHARED((_NS * _FB,), jnp.float32),
        pltpu.VMEM((_NS * _BPS,), jnp.float32),
        pltpu.VMEM((_BPS,), jnp.float32),
        pltpu.SemaphoreType.DMA,
    ],
)
def _hist_call(s_hbm, p_hbm, cnt_hbm, s_v, p_v, h_v, r_v, shared, cls_v, o_v, sem):
    cid = lax.axis_index("c")
    sid = lax.axis_index("s")
    wid = cid * _NS + sid
    col0 = wid * _COLS
    h_s = pltpu.async_copy(s_hbm.at[:, pl.ds(col0, _COLS)], s_v, sem)
    h_p = pltpu.async_copy(p_hbm.at[:, pl.ds(col0, _COLS)], p_v, sem)

    zero = jnp.zeros((_L,), jnp.float32)

    @plsc.parallel_loop(0, _HWORDS // 256, 1, unroll=2)
    def zbody(i):
        b = i * 256
        for k in range(16):
            h_v[pl.ds(b + k * _L, _L)] = zero

    h_s.wait()
    h_p.wait()

    lane = lax.broadcasted_iota(jnp.int32, (_L,), 0)
    lane_off = (lane % _PRIV) * _HSTRIDE
    mlow = lane < _PRIV
    mhigh = jnp.logical_not(mlow)

    @plsc.parallel_loop(0, _CV, 1, unroll=2)
    def mbody(j):
        b = j * _L
        for c0 in range(0, _C, 4):
            cg = range(c0, min(c0 + 4, _C))
            ss = [s_v[c, pl.ds(b, _L)] for c in cg]
            pp = [p_v[c, pl.ds(b, _L)] for c in cg]
            idxs = [lane_off +
                    (jnp.minimum((s * 128.0).astype(jnp.int32), _BINS - 1)
                     + c * _BINS)
                    for c, s in zip(cg, ss)]
            vals = [p.astype(jnp.float32) for p in pp]
            for idx, val in zip(idxs, vals):
                plsc.addupdate_scatter(h_v, [idx], val, mask=mlow)
                plsc.addupdate_scatter(h_v, [idx], val, mask=mhigh)

    @plsc.parallel_loop(0, _FB // _L, 1, unroll=4)
    def rbody(j):
        b = j * _L
        acc = h_v[pl.ds(b, _L)]
        for l in range(1, _PRIV):
            acc = acc + h_v[pl.ds(l * _HSTRIDE + b, _L)]
        r_v[pl.ds(b, _L)] = acc

    pltpu.sync_copy(r_v, shared.at[pl.ds(sid * _FB, _FB)])
    plsc.subcore_barrier()
    handles = [
        pltpu.async_copy(shared.at[pl.ds(l * _FB + sid * _BPS, _BPS)],
                         cls_v.at[pl.ds(l * _BPS, _BPS)], sem)
        for l in range(_NS)
    ]
    for h in handles:
        h.wait()

    @plsc.parallel_loop(0, _BPS // _L, 1, unroll=2)
    def cbody(k):
        b = k * _L
        acc = cls_v[pl.ds(b, _L)]
        for l in range(1, _NS):
            acc = acc + cls_v[pl.ds(l * _BPS + b, _L)]
        o_v[pl.ds(b, _L)] = acc
    pltpu.sync_copy(o_v, cnt_hbm.at[pl.ds(cid * _FB + sid * _BPS, _BPS)])


def _fit_kernel(cnt_ref, w1_ref, b1_ref, w2_ref, b2_ref, ta_ref, tb_ref):
    cnt2 = jnp.reshape(cnt_ref[...], (2 * _C, _BINS))
    cnt = cnt2[0:_C] + cnt2[_C:2 * _C]
    total = jnp.sum(cnt, axis=1, keepdims=True)
    hist = cnt / total
    h = jnp.clip(hist, 1e-6, 1.0 - 1e-6)
    h = jnp.log(h / (1.0 - h))
    h = lax.dot_general(h, w1_ref[...], (((1,), (1,)), ((), ())),
                        precision=lax.Precision.HIGHEST,
                        preferred_element_type=jnp.float32) \
        + jnp.reshape(b1_ref[...], (1, _BINS))
    h = jnp.where(h >= 0.0, h, 0.01 * h)
    d = lax.dot_general(h, w2_ref[...], (((1,), (1,)), ((), ())),
                        precision=lax.Precision.HIGHEST,
                        preferred_element_type=jnp.float32) \
        + jnp.reshape(b2_ref[...], (1, _BINS))
    mx = jnp.max(d, axis=1, keepdims=True)
    e = jnp.exp(d - mx)
    p = e / jnp.sum(e, axis=1, keepdims=True)
    rr = lax.broadcasted_iota(jnp.int32, (_BINS, _BINS), 0)
    cc = lax.broadcasted_iota(jnp.int32, (_BINS, _BINS), 1)
    tri = (rr <= cc).astype(jnp.float32)
    y = lax.dot_general(p, tri, (((1,), (0,)), ((), ())),
                        precision=lax.Precision.HIGHEST,
                        preferred_element_type=jnp.float32)
    e0 = y - p
    ji = lax.broadcasted_iota(jnp.int32, (1, _BINS), 1)
    j = ji.astype(jnp.float32)
    dxinv = jnp.where(ji == 0, 256.0, 128.0)
    x0 = jnp.where(ji == 0, 0.0, (2.0 * j - 1.0) / 256.0)
    bt = p * dxinv
    at = e0 - bt * x0
    zrows = jnp.zeros((_TROWS - _C, _BINS), jnp.float32)
    ta_ref[0:_C, :] = at
    ta_ref[_C:_TROWS, :] = zrows
    tb_ref[0:_C, :] = bt
    tb_ref[_C:_TROWS, :] = zrows


_fit_call = pl.pallas_call(
    _fit_kernel,
    out_shape=(
        jax.ShapeDtypeStruct((_TROWS, _BINS), jnp.float32),
        jax.ShapeDtypeStruct((_TROWS, _BINS), jnp.float32),
    ),
)


@functools.partial(
    pl.kernel,
    out_type=jax.ShapeDtypeStruct((_C, _BATCH), jnp.float32),
    mesh=_MESH,
    compiler_params=pltpu.CompilerParams(needs_layout_passes=False),
    scratch_types=[
        pltpu.VMEM((_C, _COLS), jnp.float32),
        pltpu.VMEM((_C, _COLS), jnp.int32),
        pltpu.VMEM((_TABN,), jnp.float32),
        pltpu.VMEM((_TABN,), jnp.float32),
        pltpu.VMEM((_C, _COLS), jnp.float32),
        pltpu.SemaphoreType.DMA,
    ],
)
def _interp_call(s_hbm, p_hbm, ta_hbm, tb_hbm, out_hbm,
                 s_v, p_v, ta_v, tb_v, o_v, sem):
    cid = lax.axis_index("c")
    sid = lax.axis_index("s")
    wid = cid * _NS + sid
    col0 = wid * _COLS

    handles = [
        pltpu.async_copy(ta_hbm, ta_v, sem),
        pltpu.async_copy(tb_hbm, tb_v, sem),
        pltpu.async_copy(s_hbm.at[:, pl.ds(col0, _COLS)], s_v, sem),
        pltpu.async_copy(p_hbm.at[:, pl.ds(col0, _COLS)], p_v, sem),
    ]
    for h in handles:
        h.wait()

    ones = jnp.ones((_L,), jnp.float32)

    @plsc.parallel_loop(0, _CV, 1, unroll=4)
    def mbody(j):
        b = j * _L
        for c0 in range(0, _C, 4):
            cg = range(c0, min(c0 + 4, _C))
            ss = [s_v[c, pl.ds(b, _L)] for c in cg]
            pp = [p_v[c, pl.ds(b, _L)] for c in cg]
            iraws = [(s * 128.0 + 0.5).astype(jnp.int32) for s in ss]
            idxs = [jnp.minimum(ir, _BINS - 1) + c * _BINS
                    for c, ir in zip(cg, iraws)]
            aa = [plsc.load_gather(ta_v, [idx]) for idx in idxs]
            bbs = [plsc.load_gather(tb_v, [idx]) for idx in idxs]
            for c, s16, p16, iraw, a, bb in zip(cg, ss, pp, iraws, aa, bbs):
                w = a + bb * s16
                y127 = a + bb * (255.0 / 256.0)
                wedge = y127 + (1.0 - y127) * (256.0 * s16 - 255.0)
                w = jnp.where(iraw >= _BINS, wedge, w)
                o_v[c, pl.ds(b, _L)] = jnp.where(p16 == 1, w, ones)
    pltpu.sync_copy(o_v, out_hbm.at[:, pl.ds(col0, _COLS)])


def kernel(y_score, y_partial, W1, b1, W2, b2):
    s_t = y_score.T
    p_t = y_partial.astype(jnp.int32).T
    cnt = _hist_call(s_t, p_t)
    ta, tb = _fit_call(cnt, W1, b1, W2, b2)
    out_t = _interp_call(s_t, p_t, ta.reshape(_TABN), tb.reshape(_TABN))
    return out_t.T


# speedup vs baseline: 1.0074x; 1.0023x over previous
import functools

import jax
import jax.numpy as jnp
from jax import lax
from jax.experimental import pallas as pl
from jax.experimental.pallas import tpu as pltpu
from jax.experimental.pallas import tpu_sc as plsc

_BINS = 128
_C = 26
_BATCH = 16384
_NC, _NS, _L = 2, 16, 16
_NW = _NC * _NS
_COLS = _BATCH // _NW
_CV = _COLS // _L
_FB = _C * _BINS
_PRIV = 8
_HSTRIDE = _FB + 1
_HWORDS = ((_PRIV * _HSTRIDE + 255) // 256) * 256
_BPS = _FB // _NS
_TROWS = 32
_TABN = _TROWS * _BINS

_MESH = plsc.VectorSubcoreMesh(core_axis_name="c", subcore_axis_name="s")


@functools.partial(
    pl.kernel,
    out_type=jax.ShapeDtypeStruct((_NC * _FB,), jnp.float32),
    mesh=_MESH,
    compiler_params=pltpu.CompilerParams(needs_layout_passes=False),
    scratch_types=[
        pltpu.VMEM((_C, _COLS), jnp.float32),
        pltpu.VMEM((_C, _COLS), jnp.int32),
        pltpu.VMEM((_HWORDS,), jnp.float32),
        pltpu.VMEM((_FB,), jnp.float32),
        pltpu.VMEM_SHARED((_NS * _FB,), jnp.float32),
        pltpu.VMEM((_NS * _BPS,), jnp.float32),
        pltpu.VMEM((_BPS,), jnp.float32),
        pltpu.SemaphoreType.DMA,
    ],
)
def _hist_call(s_hbm, p_hbm, cnt_hbm, s_v, p_v, h_v, r_v, shared, cls_v, o_v, sem):
    cid = lax.axis_index("c")
    sid = lax.axis_index("s")
    wid = cid * _NS + sid
    col0 = wid * _COLS
    h_s = pltpu.async_copy(s_hbm.at[:, pl.ds(col0, _COLS)], s_v, sem)
    h_p = pltpu.async_copy(p_hbm.at[:, pl.ds(col0, _COLS)], p_v, sem)

    zero = jnp.zeros((_L,), jnp.float32)

    @plsc.parallel_loop(0, _HWORDS // 256, 1, unroll=2)
    def zbody(i):
        b = i * 256
        for k in range(16):
            h_v[pl.ds(b + k * _L, _L)] = zero

    h_s.wait()
    h_p.wait()

    lane = lax.broadcasted_iota(jnp.int32, (_L,), 0)
    lane_off = (lane % _PRIV) * _HSTRIDE
    mlow = lane < _PRIV
    mhigh = jnp.logical_not(mlow)

    @plsc.parallel_loop(0, _CV, 1, unroll=4)
    def mbody(j):
        b = j * _L
        for c0 in range(0, _C, 4):
            cg = range(c0, min(c0 + 4, _C))
            ss = [s_v[c, pl.ds(b, _L)] for c in cg]
            pp = [p_v[c, pl.ds(b, _L)] for c in cg]
            idxs = [lane_off +
                    (jnp.minimum((s * 128.0).astype(jnp.int32), _BINS - 1)
                     + c * _BINS)
                    for c, s in zip(cg, ss)]
            vals = [p.astype(jnp.float32) for p in pp]
            for idx, val in zip(idxs, vals):
                plsc.addupdate_scatter(h_v, [idx], val, mask=mlow)
                plsc.addupdate_scatter(h_v, [idx], val, mask=mhigh)

    @plsc.parallel_loop(0, _FB // _L, 1, unroll=4)
    def rbody(j):
        b = j * _L
        acc = h_v[pl.ds(b, _L)]
        for l in range(1, _PRIV):
            acc = acc + h_v[pl.ds(l * _HSTRIDE + b, _L)]
        r_v[pl.ds(b, _L)] = acc

    pltpu.sync_copy(r_v, shared.at[pl.ds(sid * _FB, _FB)])
    plsc.subcore_barrier()
    handles = [
        pltpu.async_copy(shared.at[pl.ds(l * _FB + sid * _BPS, _BPS)],
                         cls_v.at[pl.ds(l * _BPS, _BPS)], sem)
        for l in range(_NS)
    ]
    for h in handles:
        h.wait()

    @plsc.parallel_loop(0, _BPS // _L, 1, unroll=2)
    def cbody(k):
        b = k * _L
        acc = cls_v[pl.ds(b, _L)]
        for l in range(1, _NS):
            acc = acc + cls_v[pl.ds(l * _BPS + b, _L)]
        o_v[pl.ds(b, _L)] = acc
    pltpu.sync_copy(o_v, cnt_hbm.at[pl.ds(cid * _FB + sid * _BPS, _BPS)])


def _fit_kernel(cnt_ref, w1_ref, b1_ref, w2_ref, b2_ref, ta_ref, tb_ref):
    cnt2 = jnp.reshape(cnt_ref[...], (2 * _C, _BINS))
    cnt = cnt2[0:_C] + cnt2[_C:2 * _C]
    total = jnp.sum(cnt, axis=1, keepdims=True)
    hist = cnt / total
    h = jnp.clip(hist, 1e-6, 1.0 - 1e-6)
    h = jnp.log(h / (1.0 - h))
    h = lax.dot_general(h, w1_ref[...], (((1,), (1,)), ((), ())),
                        precision=lax.Precision.HIGHEST,
                        preferred_element_type=jnp.float32) \
        + jnp.reshape(b1_ref[...], (1, _BINS))
    h = jnp.where(h >= 0.0, h, 0.01 * h)
    d = lax.dot_general(h, w2_ref[...], (((1,), (1,)), ((), ())),
                        precision=lax.Precision.HIGHEST,
                        preferred_element_type=jnp.float32) \
        + jnp.reshape(b2_ref[...], (1, _BINS))
    mx = jnp.max(d, axis=1, keepdims=True)
    e = jnp.exp(d - mx)
    p = e / jnp.sum(e, axis=1, keepdims=True)
    rr = lax.broadcasted_iota(jnp.int32, (_BINS, _BINS), 0)
    cc = lax.broadcasted_iota(jnp.int32, (_BINS, _BINS), 1)
    tri = (rr <= cc).astype(jnp.float32)
    y = lax.dot_general(p, tri, (((1,), (0,)), ((), ())),
                        precision=lax.Precision.HIGHEST,
                        preferred_element_type=jnp.float32)
    e0 = y - p
    ji = lax.broadcasted_iota(jnp.int32, (1, _BINS), 1)
    j = ji.astype(jnp.float32)
    dxinv = jnp.where(ji == 0, 256.0, 128.0)
    x0 = jnp.where(ji == 0, 0.0, (2.0 * j - 1.0) / 256.0)
    bt = p * dxinv
    at = e0 - bt * x0
    zrows = jnp.zeros((_TROWS - _C, _BINS), jnp.float32)
    ta_ref[0:_C, :] = at
    ta_ref[_C:_TROWS, :] = zrows
    tb_ref[0:_C, :] = bt
    tb_ref[_C:_TROWS, :] = zrows


_fit_call = pl.pallas_call(
    _fit_kernel,
    out_shape=(
        jax.ShapeDtypeStruct((_TROWS, _BINS), jnp.float32),
        jax.ShapeDtypeStruct((_TROWS, _BINS), jnp.float32),
    ),
)


@functools.partial(
    pl.kernel,
    out_type=jax.ShapeDtypeStruct((_C, _BATCH), jnp.float32),
    mesh=_MESH,
    compiler_params=pltpu.CompilerParams(needs_layout_passes=False),
    scratch_types=[
        pltpu.VMEM((_C, _COLS), jnp.float32),
        pltpu.VMEM((_C, _COLS), jnp.int32),
        pltpu.VMEM((_TABN,), jnp.float32),
        pltpu.VMEM((_TABN,), jnp.float32),
        pltpu.VMEM((_C, _COLS), jnp.float32),
        pltpu.SemaphoreType.DMA,
    ],
)
def _interp_call(s_hbm, p_hbm, ta_hbm, tb_hbm, out_hbm,
                 s_v, p_v, ta_v, tb_v, o_v, sem):
    cid = lax.axis_index("c")
    sid = lax.axis_index("s")
    wid = cid * _NS + sid
    col0 = wid * _COLS

    handles = [
        pltpu.async_copy(ta_hbm, ta_v, sem),
        pltpu.async_copy(tb_hbm, tb_v, sem),
        pltpu.async_copy(s_hbm.at[:, pl.ds(col0, _COLS)], s_v, sem),
        pltpu.async_copy(p_hbm.at[:, pl.ds(col0, _COLS)], p_v, sem),
    ]
    for h in handles:
        h.wait()

    ones = jnp.ones((_L,), jnp.float32)

    @plsc.parallel_loop(0, _CV, 1, unroll=4)
    def mbody(j):
        b = j * _L
        for c0 in range(0, _C, 4):
            cg = range(c0, min(c0 + 4, _C))
            ss = [s_v[c, pl.ds(b, _L)] for c in cg]
            pp = [p_v[c, pl.ds(b, _L)] for c in cg]
            iraws = [(s * 128.0 + 0.5).astype(jnp.int32) for s in ss]
            idxs = [jnp.minimum(ir, _BINS - 1) + c * _BINS
                    for c, ir in zip(cg, iraws)]
            aa = [plsc.load_gather(ta_v, [idx]) for idx in idxs]
            bbs = [plsc.load_gather(tb_v, [idx]) for idx in idxs]
            for c, s16, p16, iraw, a, bb in zip(cg, ss, pp, iraws, aa, bbs):
                w = a + bb * s16
                y127 = a + bb * (255.0 / 256.0)
                wedge = y127 + (1.0 - y127) * (256.0 * s16 - 255.0)
                w = jnp.where(iraw >= _BINS, wedge, w)
                o_v[c, pl.ds(b, _L)] = jnp.where(p16 == 1, w, ones)
    pltpu.sync_copy(o_v, out_hbm.at[:, pl.ds(col0, _COLS)])


def kernel(y_score, y_partial, W1, b1, W2, b2):
    s_t = y_score.T
    p_t = y_partial.astype(jnp.int32).T
    cnt = _hist_call(s_t, p_t)
    ta, tb = _fit_call(cnt, W1, b1, W2, b2)
    out_t = _interp_call(s_t, p_t, ta.reshape(_TABN), tb.reshape(_TABN))
    return out_t.T
